# baseline (speedup 1.0000x reference)
import jax
import jax.numpy as jnp
from jax import lax
from jax.experimental import pallas as pl
from jax.experimental.pallas import tpu as pltpu
from jax.experimental.pallas import tpu_sc as plsc

_NC, _NS, _NW, _L = 2, 16, 32, 16


def _dense_in(x, Wl, Wr, b):
    N, F = x.shape
    H = Wl.shape[1]
    BN = 1000

    def body(x_ref, wl_ref, wr_ref, b_ref, y_ref, z_ref):
        xb = x_ref[...]
        y_ref[...] = jnp.dot(xb, wl_ref[...], preferred_element_type=jnp.float32)
        z_ref[...] = (
            jnp.dot(xb, wr_ref[...], preferred_element_type=jnp.float32) + b_ref[...]
        )

    return pl.pallas_call(
        body,
        grid=(N // BN,),
        in_specs=[
            pl.BlockSpec((BN, F), lambda i: (i, 0)),
            pl.BlockSpec((F, H), lambda i: (0, 0)),
            pl.BlockSpec((F, H), lambda i: (0, 0)),
            pl.BlockSpec((1, H), lambda i: (0, 0)),
        ],
        out_specs=[
            pl.BlockSpec((BN, H), lambda i: (i, 0)),
            pl.BlockSpec((BN, H), lambda i: (i, 0)),
        ],
        out_shape=[jax.ShapeDtypeStruct((N, H), jnp.float32)] * 2,
    )(x, Wl, Wr, b.reshape(1, H))


def _seg_sum(y, edge_index, with_deg):
    N, H = y.shape
    E = edge_index.shape[1]
    NP = 10240
    RPA = NP // _NS
    RPT = NP // _NS
    EPW = E // _NW
    CH = 2000
    NCHUNK = EPW // CH

    mesh = plsc.VectorSubcoreMesh(core_axis_name="c", subcore_axis_name="s")
    out_type = [jax.ShapeDtypeStruct((_NC, NP, H), jnp.float32)]
    scratch = [
        [pltpu.VMEM((2, CH), jnp.int32)] * 2,
        [pltpu.VMEM((CH, H), jnp.float32)] * 2,
        pltpu.VMEM_SHARED((NP, H), jnp.float32),
        [pltpu.SemaphoreType.DMA] * 2,
        [pltpu.SemaphoreType.DMA] * 2,
    ]
    if with_deg:
        out_type.append(jax.ShapeDtypeStruct((_NC, NP), jnp.float32))
        scratch += [
            pltpu.VMEM((CH,), jnp.float32),
            pltpu.VMEM_SHARED((NP,), jnp.float32),
        ]

    def body(y_hbm, ei_hbm, z2d_hbm, z1d_hbm, *rest):
        if with_deg:
            (out_hbm, deg_hbm, idx2, rows, aggsh, isem, gsem, ones, degsh) = rest
        else:
            (out_hbm, idx2, rows, aggsh, isem, gsem) = rest
        c = lax.axis_index("c")
        s = lax.axis_index("s")
        w = c * _NS + s

        pltpu.sync_copy(z2d_hbm, aggsh.at[pl.ds(s * RPA, RPA)])
        if with_deg:
            pltpu.sync_copy(z1d_hbm, degsh.at[pl.ds(s * RPT, RPT)])

            def ofill(i, carry):
                ones[pl.ds(i * _L, _L)] = jnp.full((_L,), 1.0, jnp.float32)
                return carry

            lax.fori_loop(0, CH // _L, ofill, 0)
        plsc.subcore_barrier()

        base = w * EPW
        idxcp = [None, None]
        gcp = [None, None]
        idxcp[0] = pltpu.async_copy(ei_hbm.at[:, pl.ds(base, CH)], idx2[0], isem[0])
        for k in range(NCHUNK):
            b = k & 1
            idxcp[b].wait()
            gcp[b] = pltpu.async_copy(y_hbm.at[idx2[b].at[0]], rows[b], gsem[b])
            if k > 0:
                pb = 1 - b
                gcp[pb].wait()
                pltpu.sync_copy(rows[pb], aggsh.at[idx2[pb].at[1]], add=True)
                if with_deg:
                    pltpu.sync_copy(ones, degsh.at[idx2[pb].at[1]], add=True)
            if k + 1 < NCHUNK:
                idxcp[1 - b] = pltpu.async_copy(
                    ei_hbm.at[:, pl.ds(base + (k + 1) * CH, CH)],
                    idx2[1 - b],
                    isem[1 - b],
                )
        lb = (NCHUNK - 1) & 1
        gcp[lb].wait()
        pltpu.sync_copy(rows[lb], aggsh.at[idx2[lb].at[1]], add=True)
        if with_deg:
            pltpu.sync_copy(ones, degsh.at[idx2[lb].at[1]], add=True)
        plsc.subcore_barrier()
        pltpu.sync_copy(
            aggsh.at[pl.ds(s * RPA, RPA)], out_hbm.at[c, pl.ds(s * RPA, RPA)]
        )
        if with_deg:
            pltpu.sync_copy(
                degsh.at[pl.ds(s * RPT, RPT)], deg_hbm.at[c, pl.ds(s * RPT, RPT)]
            )

    fn = pl.kernel(
        body,
        out_type=out_type,
        mesh=mesh,
        scratch_types=scratch,
        compiler_params=pltpu.CompilerParams(
            use_tc_tiling_on_sc=False, needs_layout_passes=False
        ),
    )
    z2d = jnp.zeros((RPA, H), jnp.float32)
    z1d = jnp.zeros((RPT,), jnp.float32)
    return fn(y, edge_index, z2d, z1d)


def _mega(agg1p, degp, z1p, Wz, b2, Wcp, bcp, edge_index, N):
    NP, H = z1p.shape
    E = edge_index.shape[1]
    C2 = 8
    RPT = NP // _NS
    BLK = 160
    NBLK = RPT // BLK
    ES = E // _NS
    CHS = 800
    NCHS = ES // CHS
    EPW = E // _NW
    CHE = 2000
    NCHE = EPW // CHE
    NG = CHE // _L
    U = 5

    mesh = plsc.VectorSubcoreMesh(core_axis_name="c", subcore_axis_name="s")
    out_type = [
        jax.ShapeDtypeStruct((E, 4), jnp.float32),
        jax.ShapeDtypeStruct((_NC, NP, H), jnp.float32),
        jax.ShapeDtypeStruct((_NC, NP, H), jnp.float32),
        jax.ShapeDtypeStruct((_NC, NP, C2), jnp.float32),
    ]
    scratch = [
        pltpu.VMEM_SHARED((NP, H), jnp.float32),
        pltpu.VMEM((CHE, C2), jnp.float32),
        pltpu.VMEM((CHE, C2), jnp.float32),
        [pltpu.VMEM((2, CHS), jnp.int32)] * 2,
        [pltpu.VMEM((CHS, H), jnp.float32)] * 2,
        [pltpu.VMEM((2, CHE), jnp.int32)] * 2,
        pltpu.VMEM((CHE, 4), jnp.float32),
        pltpu.VMEM((2 * H, H), jnp.float32),
        pltpu.VMEM((H, H), jnp.float32),
        pltpu.VMEM((H,), jnp.float32),
        pltpu.VMEM((H,), jnp.float32),
        pltpu.VMEM((_L,), jnp.float32),
        pltpu.VMEM((160,), jnp.float32),
        pltpu.VMEM((160,), jnp.float32),
        pltpu.VMEM((160, C2), jnp.float32),
        [pltpu.SemaphoreType.DMA] * 2,
        [pltpu.SemaphoreType.DMA] * 2,
        [pltpu.SemaphoreType.DMA] * 2,
        [pltpu.SemaphoreType.DMA] * 2,
    ]

    def body(a1_hbm, dg_hbm, z1_hbm, wz_hbm, b2_hbm, wc_hbm, bc_hbm, ei_hbm,
             z2d_hbm, out_hbm, y2c_hbm, z2c_hbm, tc_hbm, aggsh, gbufa, gbufb,
             idx2, rows, eidx, outb, wzb, wcb, b2b, bcb, hbuf, db0, db1, tblk,
             isem, gsem, eisem, osem):
        c = lax.axis_index("c")
        s = lax.axis_index("s")
        w = c * _NS + s
        r0 = rows[0]

        pltpu.sync_copy(wz_hbm, wzb)
        pltpu.sync_copy(wc_hbm, wcb)
        pltpu.sync_copy(b2_hbm, b2b)
        pltpu.sync_copy(bc_hbm, bcb)
        pltpu.sync_copy(z2d_hbm, aggsh.at[pl.ds(s * RPT, RPT)])

        lanes = lax.iota(jnp.int32, _L)
        zl = jnp.zeros((_L,), jnp.int32)
        kidx = [jnp.full((_L,), k, jnp.int32) for k in range(H)]

        wrl = [wzb[k, :] for k in range(H)]
        wrr = [wzb[H + k, :] for k in range(H)]
        b2v = b2b[...]
        BLK = 160
        for blk in range(NBLK):
            rw = s * RPT + blk * BLK
            pltpu.sync_copy(a1_hbm.at[0, pl.ds(rw, BLK)], r0.at[pl.ds(0, BLK)])
            pltpu.sync_copy(a1_hbm.at[1, pl.ds(rw, BLK)], r0.at[pl.ds(BLK, BLK)])
            pltpu.sync_copy(z1_hbm.at[pl.ds(rw, BLK)], r0.at[pl.ds(2 * BLK, BLK)])
            pltpu.sync_copy(dg_hbm.at[0, pl.ds(rw, BLK)], db0)
            pltpu.sync_copy(dg_hbm.at[1, pl.ds(rw, BLK)], db1)

            def node1(i, carry):
                d0v = plsc.load_gather(db0, [zl + i])
                d1v = plsc.load_gather(db1, [zl + i])
                inv = 1.0 / jnp.maximum(d0v + d1v, 1.0)
                h = jnp.maximum(
                    (r0[i, :] + r0[BLK + i, :]) * inv + r0[2 * BLK + i, :], 0.0
                )
                acc_y = jnp.zeros((_L,), jnp.float32)
                acc_z = b2v
                for k in range(H):
                    hk = h.at[kidx[k]].get(mode="promise_in_bounds")
                    acc_y = acc_y + hk * wrl[k]
                    acc_z = acc_z + hk * wrr[k]
                r0[3 * BLK + i, :] = acc_y
                r0[4 * BLK + i, :] = acc_z
                return carry

            lax.fori_loop(0, BLK, node1, 0)
            pltpu.sync_copy(r0.at[pl.ds(3 * BLK, BLK)], y2c_hbm.at[c, pl.ds(rw, BLK)])
            pltpu.sync_copy(r0.at[pl.ds(4 * BLK, BLK)], z2c_hbm.at[c, pl.ds(rw, BLK)])
        plsc.subcore_barrier()

        base = s * ES
        idxcp = [None, None]
        gcp = [None, None]
        idxcp[0] = pltpu.async_copy(ei_hbm.at[:, pl.ds(base, CHS)], idx2[0], isem[0])
        for k in range(NCHS):
            b = k & 1
            idxcp[b].wait()
            gcp[b] = pltpu.async_copy(
                y2c_hbm.at[c].at[idx2[b].at[0]], rows[b], gsem[b]
            )
            if k > 0:
                pb = 1 - b
                gcp[pb].wait()
                pltpu.sync_copy(rows[pb], aggsh.at[idx2[pb].at[1]], add=True)
            if k + 1 < NCHS:
                idxcp[1 - b] = pltpu.async_copy(
                    ei_hbm.at[:, pl.ds(base + (k + 1) * CHS, CHS)],
                    idx2[1 - b],
                    isem[1 - b],
                )
        lb = (NCHS - 1) & 1
        gcp[lb].wait()
        pltpu.sync_copy(rows[lb], aggsh.at[idx2[lb].at[1]], add=True)
        plsc.subcore_barrier()

        wcr = [wcb[k, :] for k in range(H)]
        bcv = bcb[...]
        lmask = lanes < C2
        for blk in range(NBLK):
            rw = s * RPT + blk * BLK
            pltpu.sync_copy(aggsh.at[pl.ds(rw, BLK)], r0.at[pl.ds(0, BLK)])
            pltpu.sync_copy(z2c_hbm.at[c, pl.ds(rw, BLK)], r0.at[pl.ds(BLK, BLK)])
            pltpu.sync_copy(dg_hbm.at[0, pl.ds(rw, BLK)], db0)
            pltpu.sync_copy(dg_hbm.at[1, pl.ds(rw, BLK)], db1)

            def node2(i, carry):
                d0v = plsc.load_gather(db0, [zl + i])
                d1v = plsc.load_gather(db1, [zl + i])
                inv = 1.0 / jnp.maximum(d0v + d1v, 1.0)
                h = jnp.maximum(r0[i, :] * inv + r0[BLK + i, :], 0.0)
                acc = bcv
                for k in range(H):
                    hk = h.at[kidx[k]].get(mode="promise_in_bounds")
                    acc = acc + hk * wcr[k]
                plsc.store_scatter(tblk, [zl + i, lanes], acc, mask=lmask)
                return carry

            lax.fori_loop(0, BLK, node2, 0)
            pltpu.sync_copy(tblk, tc_hbm.at[c, pl.ds(rw, BLK)])
        plsc.subcore_barrier()

        ebase = w * EPW
        cols = [jnp.full((_L,), j, jnp.int32) for j in range(C2)]
        eicp = [None, None]
        ocp = None
        eicp[0] = pltpu.async_copy(ei_hbm.at[:, pl.ds(ebase, CHE)], eidx[0], eisem[0])
        for k in range(NCHE):
            b = k & 1
            eicp[b].wait()
            ga = pltpu.async_copy(
                tc_hbm.at[c].at[eidx[b].at[0]], gbufa, gsem[0]
            )
            gb = pltpu.async_copy(
                tc_hbm.at[c].at[eidx[b].at[1]], gbufb, gsem[1]
            )
            if k + 1 < NCHE:
                eicp[1 - b] = pltpu.async_copy(
                    ei_hbm.at[:, pl.ds(ebase + (k + 1) * CHE, CHE)],
                    eidx[1 - b],
                    eisem[1 - b],
                )
            ga.wait()
            gb.wait()
            if ocp is not None:
                ocp.wait()
            ob = outb

            def group(g0, carry):
                for u in range(U):
                    g = g0 * U + u
                    pos = lanes + g * _L
                    v = []
                    for j in range(4):
                        av = plsc.load_gather(gbufa, [pos, cols[j]])
                        bv = plsc.load_gather(gbufb, [pos, cols[4 + j]])
                        v.append(av + bv)
                    m = jnp.maximum(jnp.maximum(v[0], v[1]), jnp.maximum(v[2], v[3]))
                    ex = [jnp.exp(vj - m) for vj in v]
                    ssum = ex[0] + ex[1] + ex[2] + ex[3]
                    t = 2.0 * (ssum - 1.0) / (ssum + 1.0)
                    t = t - 1.0 + ssum * jnp.exp(-t)
                    t = t - 1.0 + ssum * jnp.exp(-t)
                    t = t - 1.0 + ssum * jnp.exp(-t)
                    for j in range(4):
                        plsc.store_scatter(ob, [pos, cols[j]], v[j] - m - t)
                return carry

            lax.fori_loop(0, NG // U, group, 0)
            ocp = pltpu.async_copy(
                ob, out_hbm.at[pl.ds(ebase + k * CHE, CHE)], osem[0]
            )
        ocp.wait()

    fn = pl.kernel(
        body,
        out_type=out_type,
        mesh=mesh,
        scratch_types=scratch,
        compiler_params=pltpu.CompilerParams(
            use_tc_tiling_on_sc=False, needs_layout_passes=False
        ),
    )
    z2d = jnp.zeros((RPT, H), jnp.float32)
    return fn(agg1p, degp, z1p, Wz, b2, Wcp, bcp, edge_index, z2d)


def kernel(x, edge_index, W1_l, W1_r, b1, W2_l, W2_r, b2, Wc, bc):
    N = x.shape[0]
    H = W1_l.shape[1]
    C = Wc.shape[1]
    NP = 10240

    y1, z1 = _dense_in(x, W1_l, W1_r, b1)
    agg1p, degp = _seg_sum(y1, edge_index, with_deg=True)
    z1p = jnp.pad(z1, ((0, NP - N), (0, 0)))
    Wz = jnp.concatenate([W2_l, W2_r], axis=0)
    Wcp = jnp.zeros((H, H), jnp.float32)
    Wcp = Wcp.at[:, :C].set(Wc[:H]).at[:, C : 2 * C].set(Wc[H:])
    bcp = jnp.zeros((H,), jnp.float32).at[:C].set(bc)
    out, _, _, _ = _mega(agg1p, degp, z1p, Wz, b2, Wcp, bcp, edge_index, N)
    return out
